# baseline (speedup 1.0000x reference)
import functools

import jax
import jax.numpy as jnp
from jax import lax
from jax.experimental import pallas as pl
from jax.experimental.pallas import tpu as pltpu
from jax.experimental.pallas import tpu_sc as plsc

N_NODES = 10000
N_EDGES = 320000
D_EDGE = 16
D_NODE = 128
D_GLOBAL = 64
N_GRAPHS = 16

NC = 2
NS = 16
NW = NC * NS
SUB = 128
IDX_ROWS = N_EDGES // SUB
ROWS_PER_TILE = IDX_ROWS // NW
ROWS_PER_CHUNK = 6
NCHUNKS = ROWS_PER_TILE // ROWS_PER_CHUNK
CHUNK = SUB * ROWS_PER_CHUNK
N_TAIL = IDX_ROWS - ROWS_PER_TILE * NW

NODE_BLK = 1000
N_BLKS = N_NODES // NODE_BLK


def _sc_scatter_body(ea4_hbm, ei3_hbm, out_s_hbm, out_c_hbm,
                     vals_v, ch_v, idx_v, ones_v, acc_sh, cnt_sh, sem):
    c = lax.axis_index("c")
    s = lax.axis_index("s")
    w = s * NC + c
    i32 = jnp.int32

    zero16 = jnp.zeros((16,), jnp.float32)
    one16 = jnp.ones((16,), jnp.float32)

    def zrow(i, carry):
        vals_v[i, :] = zero16
        return carry

    lax.fori_loop(0, NODE_BLK, zrow, 0)

    def fill1(i, carry):
        ones_v[pl.ds(i * 16, 16)] = zero16
        return carry

    lax.fori_loop(0, NODE_BLK // 16, fill1, 0)

    @pl.when(s < N_BLKS)
    def _zero_shared():
        pltpu.sync_copy(vals_v.at[pl.ds(0, NODE_BLK)],
                        acc_sh.at[pl.ds(s * NODE_BLK, NODE_BLK)])
        pltpu.sync_copy(ones_v.at[pl.ds(0, NODE_BLK)],
                        cnt_sh.at[pl.ds(s * NODE_BLK, NODE_BLK)])

    def fill2(i, carry):
        ones_v[pl.ds(i * 16, 16)] = one16
        return carry

    lax.fori_loop(0, NODE_BLK // 16, fill2, 0)

    plsc.subcore_barrier()

    PITCH = SUB + 1
    lane = lax.iota(i32, 16)
    one_v = jnp.full((16,), 1, i32)

    UNROLL = 8

    def transpose_blocks(nblocks):
        rvs = tuple(lane + jnp.full((16,), b * 16, i32)
                    for b in range(nblocks))

        def ebody(eu, cvec):
            e0 = eu * UNROLL
            gs = []
            cv = cvec
            for u in range(UNROLL):
                for b in range(nblocks):
                    gs.append(plsc.load_gather(ch_v, [rvs[b], cv]))
                cv = cv + one_v
            i = 0
            for u in range(UNROLL):
                for b in range(nblocks):
                    vals_v[b * SUB + e0 + u, :] = gs[i]
                    i += 1
            return cv

        lax.fori_loop(0, SUB // UNROLL, ebody, jnp.full((16,), 0, i32))

    def chunk_body(k, carry):
        row0 = w * ROWS_PER_TILE + k * ROWS_PER_CHUNK
        loads = [
            pltpu.async_copy(ei3_hbm.at[pl.ds(row0, ROWS_PER_CHUNK), 0],
                             idx_v, sem),
        ]
        for b in range(ROWS_PER_CHUNK):
            for tf in range(2):
                loads.append(pltpu.async_copy(
                    ea4_hbm.at[tf, row0 + b],
                    ch_v.at[pl.ds(b * 16 + tf * 8, 8), pl.ds(0, SUB)], sem))
        for d in loads:
            d.wait()
        transpose_blocks(ROWS_PER_CHUNK)
        scats = [pltpu.async_copy(vals_v.at[pl.ds(j * SUB, SUB)],
                                  acc_sh.at[idx_v.at[j]], sem, add=True)
                 for j in range(ROWS_PER_CHUNK)]
        scats += [pltpu.async_copy(ones_v.at[pl.ds(j * SUB, SUB)],
                                   cnt_sh.at[idx_v.at[j]], sem, add=True)
                  for j in range(ROWS_PER_CHUNK)]
        for d in scats:
            d.wait()
        return carry

    lax.fori_loop(0, NCHUNKS, chunk_body, 0)

    @pl.when(w < N_TAIL)
    def _tail():
        row = ROWS_PER_TILE * NW + w
        pltpu.sync_copy(ei3_hbm.at[row, 0], idx_v.at[0])
        pltpu.sync_copy(ea4_hbm.at[0, row],
                        ch_v.at[pl.ds(0, 8), pl.ds(0, SUB)])
        pltpu.sync_copy(ea4_hbm.at[1, row],
                        ch_v.at[pl.ds(8, 8), pl.ds(0, SUB)])
        transpose_blocks(1)
        pltpu.sync_copy(vals_v.at[pl.ds(0, SUB)],
                        acc_sh.at[idx_v.at[0]], add=True)
        pltpu.sync_copy(ones_v.at[pl.ds(0, SUB)],
                        cnt_sh.at[idx_v.at[0]], add=True)

    plsc.subcore_barrier()

    @pl.when(s < N_BLKS)
    def _writeback():
        pltpu.sync_copy(acc_sh.at[pl.ds(s * NODE_BLK, NODE_BLK)],
                        vals_v.at[pl.ds(0, NODE_BLK)])
        pltpu.sync_copy(vals_v.at[pl.ds(0, NODE_BLK)],
                        out_s_hbm.at[c, pl.ds(s * NODE_BLK, NODE_BLK)])
        pltpu.sync_copy(cnt_sh.at[pl.ds(s * NODE_BLK, NODE_BLK)],
                        ones_v.at[pl.ds(0, NODE_BLK)])
        pltpu.sync_copy(ones_v.at[pl.ds(0, NODE_BLK)],
                        out_c_hbm.at[c, pl.ds(s * NODE_BLK, NODE_BLK)])


@functools.lru_cache(maxsize=1)
def _sc_scatter():
    return pl.kernel(
        _sc_scatter_body,
        out_type=(jax.ShapeDtypeStruct((NC, N_NODES, D_EDGE), jnp.float32),
                  jax.ShapeDtypeStruct((NC, N_NODES), jnp.float32)),
        mesh=plsc.VectorSubcoreMesh(core_axis_name="c", subcore_axis_name="s",
                                    num_cores=NC, num_subcores=NS),
        compiler_params=pltpu.CompilerParams(use_tc_tiling_on_sc=False,
                                             needs_layout_passes=False),
        scratch_types=[
            pltpu.VMEM((NODE_BLK, D_EDGE), jnp.float32),
            pltpu.VMEM((ROWS_PER_CHUNK * 16, SUB + 1), jnp.float32),
            pltpu.VMEM((ROWS_PER_CHUNK, SUB), jnp.int32),
            pltpu.VMEM((NODE_BLK,), jnp.float32),
            pltpu.VMEM_SHARED((N_NODES, D_EDGE), jnp.float32),
            pltpu.VMEM_SHARED((N_NODES,), jnp.float32),
            pltpu.SemaphoreType.DMA,
        ],
    )


def _tc_mlp_body(x_ref, p_ref, c_ref, b_ref, u_ref, w1x_ref, w1e_ref,
                 w1u_ref, b1_ref, w2_ref, b2_ref, out_ref):
    f32 = jnp.float32
    p = p_ref[...]
    sums = p[0] + p[1]
    cnt = c_ref[...]
    denom = jnp.maximum(cnt[:, 0:1] + cnt[:, 1:2], 1.0)
    agg = sums / denom

    bvec = b_ref[0]
    ids = lax.broadcasted_iota(jnp.int32, (N_GRAPHS, NODE_BLK), 0)
    oht = (ids == jnp.broadcast_to(bvec, (N_GRAPHS, NODE_BLK))).astype(f32)

    uw = jnp.dot(u_ref[...], w1u_ref[...], preferred_element_type=f32)
    u_contrib = lax.dot_general(oht, uw, (((0,), (0,)), ((), ())),
                                preferred_element_type=f32)

    pre = (jnp.dot(x_ref[...], w1x_ref[...], preferred_element_type=f32)
           + jnp.dot(agg, w1e_ref[...], preferred_element_type=f32)
           + u_contrib + b1_ref[...])
    h = jnp.maximum(pre, 0.0)
    out_ref[...] = jnp.dot(h, w2_ref[...], preferred_element_type=f32) + b2_ref[...]


def kernel(x, edge_index, edge_attr, u, batch, W1, b1, W2, b2):
    f32 = jnp.float32
    ea4 = (edge_attr.astype(f32).T.reshape(2, 8, IDX_ROWS, SUB)
           .transpose(0, 2, 1, 3))
    ei3 = (edge_index.astype(jnp.int32).transpose(1, 0)
           .reshape(IDX_ROWS, SUB, 2).transpose(0, 2, 1))

    sums2, cnt2 = _sc_scatter()(ea4, ei3)
    cnt_t = cnt2.T

    batch3 = batch.astype(jnp.int32).reshape(N_BLKS, 1, NODE_BLK)
    W1x = W1[:D_NODE]
    W1e = W1[D_NODE:D_NODE + D_EDGE]
    W1u = W1[D_NODE + D_EDGE:]
    b1r = b1.reshape(1, -1)
    b2r = b2.reshape(1, -1)

    out = pl.pallas_call(
        _tc_mlp_body,
        grid=(N_BLKS,),
        in_specs=[
            pl.BlockSpec((NODE_BLK, D_NODE), lambda i: (i, 0)),
            pl.BlockSpec((NC, NODE_BLK, D_EDGE), lambda i: (0, i, 0)),
            pl.BlockSpec((NODE_BLK, NC), lambda i: (i, 0)),
            pl.BlockSpec((1, 1, NODE_BLK), lambda i: (i, 0, 0)),
            pl.BlockSpec((N_GRAPHS, D_GLOBAL), lambda i: (0, 0)),
            pl.BlockSpec((D_NODE, 128), lambda i: (0, 0)),
            pl.BlockSpec((D_EDGE, 128), lambda i: (0, 0)),
            pl.BlockSpec((D_GLOBAL, 128), lambda i: (0, 0)),
            pl.BlockSpec((1, 128), lambda i: (0, 0)),
            pl.BlockSpec((128, 128), lambda i: (0, 0)),
            pl.BlockSpec((1, 128), lambda i: (0, 0)),
        ],
        out_specs=pl.BlockSpec((NODE_BLK, 128), lambda i: (i, 0)),
        out_shape=jax.ShapeDtypeStruct((N_NODES, 128), f32),
    )(x, sums2, cnt_t, batch3, u, W1x, W1e, W1u, b1r, W2, b2r)
    return out


# speedup vs baseline: 1.0674x; 1.0674x over previous
import functools

import jax
import jax.numpy as jnp
from jax import lax
from jax.experimental import pallas as pl
from jax.experimental.pallas import tpu as pltpu
from jax.experimental.pallas import tpu_sc as plsc

N_NODES = 10000
N_EDGES = 320000
D_EDGE = 16
D_NODE = 128
D_GLOBAL = 64
N_GRAPHS = 16

NC = 2
NS = 16
NW = NC * NS
SUB = 128
IDX_ROWS = N_EDGES // SUB
ROWS_PER_TILE = IDX_ROWS // NW
ROWS_PER_CHUNK = 6
NCHUNKS = ROWS_PER_TILE // ROWS_PER_CHUNK
CHUNK = SUB * ROWS_PER_CHUNK
N_TAIL = IDX_ROWS - ROWS_PER_TILE * NW

NODE_BLK = 1000
N_BLKS = N_NODES // NODE_BLK


def _sc_scatter_body(ea4_hbm, ei3_hbm, out_s_hbm, out_c_hbm,
                     vals_v, ch_v, idx_v, ones_v, acc_sh, cnt_sh, sem):
    c = lax.axis_index("c")
    s = lax.axis_index("s")
    w = s * NC + c
    i32 = jnp.int32

    zero16 = jnp.zeros((16,), jnp.float32)
    one16 = jnp.ones((16,), jnp.float32)

    def zrow(i, carry):
        vals_v[i, :] = zero16
        return carry

    lax.fori_loop(0, NODE_BLK, zrow, 0)

    def fill1(i, carry):
        ones_v[pl.ds(i * 16, 16)] = zero16
        return carry

    lax.fori_loop(0, NODE_BLK // 16, fill1, 0)

    @pl.when(s < N_BLKS)
    def _zero_shared():
        pltpu.sync_copy(vals_v.at[pl.ds(0, NODE_BLK)],
                        acc_sh.at[pl.ds(s * NODE_BLK, NODE_BLK)])
        pltpu.sync_copy(ones_v.at[pl.ds(0, NODE_BLK)],
                        cnt_sh.at[pl.ds(s * NODE_BLK, NODE_BLK)])

    def fill2(i, carry):
        ones_v[pl.ds(i * 16, 16)] = one16
        return carry

    lax.fori_loop(0, NODE_BLK // 16, fill2, 0)

    plsc.subcore_barrier()

    PITCH = SUB + 1
    lane = lax.iota(i32, 16)
    one_v = jnp.full((16,), 1, i32)

    UNROLL = 4

    def transpose_blocks(nblocks):
        rvs = tuple(lane + jnp.full((16,), b * 16, i32)
                    for b in range(nblocks))

        def ebody(eu, cvec):
            e0 = eu * UNROLL
            gs = []
            cv = cvec
            for u in range(UNROLL):
                for b in range(nblocks):
                    gs.append(plsc.load_gather(ch_v, [rvs[b], cv]))
                cv = cv + one_v
            i = 0
            for u in range(UNROLL):
                for b in range(nblocks):
                    vals_v[b * SUB + e0 + u, :] = gs[i]
                    i += 1
            return cv

        lax.fori_loop(0, SUB // UNROLL, ebody, jnp.full((16,), 0, i32))

    def chunk_body(k, carry):
        row0 = w * ROWS_PER_TILE + k * ROWS_PER_CHUNK
        loads = [
            pltpu.async_copy(ei3_hbm.at[pl.ds(row0, ROWS_PER_CHUNK), 0],
                             idx_v, sem),
        ]
        for b in range(ROWS_PER_CHUNK):
            for tf in range(2):
                loads.append(pltpu.async_copy(
                    ea4_hbm.at[tf, row0 + b],
                    ch_v.at[pl.ds(b * 16 + tf * 8, 8), pl.ds(0, SUB)], sem))
        for d in loads:
            d.wait()
        transpose_blocks(ROWS_PER_CHUNK)
        scats = [pltpu.async_copy(vals_v.at[pl.ds(j * SUB, SUB)],
                                  acc_sh.at[idx_v.at[j]], sem, add=True)
                 for j in range(ROWS_PER_CHUNK)]
        scats += [pltpu.async_copy(ones_v.at[pl.ds(j * SUB, SUB)],
                                   cnt_sh.at[idx_v.at[j]], sem, add=True)
                  for j in range(ROWS_PER_CHUNK)]
        for d in scats:
            d.wait()
        return carry

    lax.fori_loop(0, NCHUNKS, chunk_body, 0)

    @pl.when(w < N_TAIL)
    def _tail():
        row = ROWS_PER_TILE * NW + w
        pltpu.sync_copy(ei3_hbm.at[row, 0], idx_v.at[0])
        pltpu.sync_copy(ea4_hbm.at[0, row],
                        ch_v.at[pl.ds(0, 8), pl.ds(0, SUB)])
        pltpu.sync_copy(ea4_hbm.at[1, row],
                        ch_v.at[pl.ds(8, 8), pl.ds(0, SUB)])
        transpose_blocks(1)
        pltpu.sync_copy(vals_v.at[pl.ds(0, SUB)],
                        acc_sh.at[idx_v.at[0]], add=True)
        pltpu.sync_copy(ones_v.at[pl.ds(0, SUB)],
                        cnt_sh.at[idx_v.at[0]], add=True)

    plsc.subcore_barrier()

    @pl.when(s < N_BLKS)
    def _writeback():
        pltpu.sync_copy(acc_sh.at[pl.ds(s * NODE_BLK, NODE_BLK)],
                        vals_v.at[pl.ds(0, NODE_BLK)])
        pltpu.sync_copy(vals_v.at[pl.ds(0, NODE_BLK)],
                        out_s_hbm.at[c, pl.ds(s * NODE_BLK, NODE_BLK)])
        pltpu.sync_copy(cnt_sh.at[pl.ds(s * NODE_BLK, NODE_BLK)],
                        ones_v.at[pl.ds(0, NODE_BLK)])
        pltpu.sync_copy(ones_v.at[pl.ds(0, NODE_BLK)],
                        out_c_hbm.at[c, pl.ds(s * NODE_BLK, NODE_BLK)])


@functools.lru_cache(maxsize=1)
def _sc_scatter():
    return pl.kernel(
        _sc_scatter_body,
        out_type=(jax.ShapeDtypeStruct((NC, N_NODES, D_EDGE), jnp.float32),
                  jax.ShapeDtypeStruct((NC, N_NODES), jnp.float32)),
        mesh=plsc.VectorSubcoreMesh(core_axis_name="c", subcore_axis_name="s",
                                    num_cores=NC, num_subcores=NS),
        compiler_params=pltpu.CompilerParams(use_tc_tiling_on_sc=False,
                                             needs_layout_passes=False),
        scratch_types=[
            pltpu.VMEM((NODE_BLK, D_EDGE), jnp.float32),
            pltpu.VMEM((ROWS_PER_CHUNK * 16, SUB + 1), jnp.float32),
            pltpu.VMEM((ROWS_PER_CHUNK, SUB), jnp.int32),
            pltpu.VMEM((NODE_BLK,), jnp.float32),
            pltpu.VMEM_SHARED((N_NODES, D_EDGE), jnp.float32),
            pltpu.VMEM_SHARED((N_NODES,), jnp.float32),
            pltpu.SemaphoreType.DMA,
        ],
    )


def _tc_mlp_body(x_ref, p_ref, c_ref, b_ref, u_ref, w1x_ref, w1e_ref,
                 w1u_ref, b1_ref, w2_ref, b2_ref, out_ref):
    f32 = jnp.float32
    p = p_ref[...]
    sums = p[0] + p[1]
    cnt = c_ref[...]
    denom = jnp.maximum(cnt[:, 0:1] + cnt[:, 1:2], 1.0)
    agg = sums / denom

    bvec = b_ref[0]
    ids = lax.broadcasted_iota(jnp.int32, (N_GRAPHS, NODE_BLK), 0)
    oht = (ids == jnp.broadcast_to(bvec, (N_GRAPHS, NODE_BLK))).astype(f32)

    uw = jnp.dot(u_ref[...], w1u_ref[...], preferred_element_type=f32)
    u_contrib = lax.dot_general(oht, uw, (((0,), (0,)), ((), ())),
                                preferred_element_type=f32)

    pre = (jnp.dot(x_ref[...], w1x_ref[...], preferred_element_type=f32)
           + jnp.dot(agg, w1e_ref[...], preferred_element_type=f32)
           + u_contrib + b1_ref[...])
    h = jnp.maximum(pre, 0.0)
    out_ref[...] = jnp.dot(h, w2_ref[...], preferred_element_type=f32) + b2_ref[...]


def kernel(x, edge_index, edge_attr, u, batch, W1, b1, W2, b2):
    f32 = jnp.float32
    ea4 = (edge_attr.astype(f32).T.reshape(2, 8, IDX_ROWS, SUB)
           .transpose(0, 2, 1, 3))
    ei3 = (edge_index.astype(jnp.int32).transpose(1, 0)
           .reshape(IDX_ROWS, SUB, 2).transpose(0, 2, 1))

    sums2, cnt2 = _sc_scatter()(ea4, ei3)
    cnt_t = cnt2.T

    batch3 = batch.astype(jnp.int32).reshape(N_BLKS, 1, NODE_BLK)
    W1x = W1[:D_NODE]
    W1e = W1[D_NODE:D_NODE + D_EDGE]
    W1u = W1[D_NODE + D_EDGE:]
    b1r = b1.reshape(1, -1)
    b2r = b2.reshape(1, -1)

    out = pl.pallas_call(
        _tc_mlp_body,
        grid=(N_BLKS,),
        in_specs=[
            pl.BlockSpec((NODE_BLK, D_NODE), lambda i: (i, 0)),
            pl.BlockSpec((NC, NODE_BLK, D_EDGE), lambda i: (0, i, 0)),
            pl.BlockSpec((NODE_BLK, NC), lambda i: (i, 0)),
            pl.BlockSpec((1, 1, NODE_BLK), lambda i: (i, 0, 0)),
            pl.BlockSpec((N_GRAPHS, D_GLOBAL), lambda i: (0, 0)),
            pl.BlockSpec((D_NODE, 128), lambda i: (0, 0)),
            pl.BlockSpec((D_EDGE, 128), lambda i: (0, 0)),
            pl.BlockSpec((D_GLOBAL, 128), lambda i: (0, 0)),
            pl.BlockSpec((1, 128), lambda i: (0, 0)),
            pl.BlockSpec((128, 128), lambda i: (0, 0)),
            pl.BlockSpec((1, 128), lambda i: (0, 0)),
        ],
        out_specs=pl.BlockSpec((NODE_BLK, 128), lambda i: (i, 0)),
        out_shape=jax.ShapeDtypeStruct((N_NODES, 128), f32),
    )(x, sums2, cnt_t, batch3, u, W1x, W1e, W1u, b1r, W2, b2r)
    return out
